# baseline (speedup 1.0000x reference)
import functools

import jax
import jax.numpy as jnp
from jax import lax
from jax.experimental import pallas as pl
from jax.experimental.pallas import tpu as pltpu
from jax.experimental.pallas import tpu_sc as plsc


def _sc_plane(row_embed, col_embed, hw, d):
    info = plsc.get_sparse_core_info()
    nc, ns = info.num_cores, info.num_subcores
    nw = nc * ns
    rows = hw // nw
    mesh = plsc.VectorSubcoreMesh(core_axis_name="c", subcore_axis_name="s")

    @functools.partial(
        pl.kernel,
        out_type=jax.ShapeDtypeStruct((hw, 2 * d), jnp.float32),
        mesh=mesh,
        scratch_types=[
            pltpu.VMEM((rows, 2 * d), jnp.float32),
            pltpu.SemaphoreType.DMA,
        ],
    )
    def pos_plane_kernel(row_hbm, col_hbm, plane_hbm, plane_v, sem):
        wid = lax.axis_index("s") * nc + lax.axis_index("c")
        fills = []
        for r in range(rows):
            fills.append(pltpu.async_copy(
                col_hbm.at[pl.ds(r, 1)],
                plane_v.at[pl.ds(r, 1), pl.ds(0, d)], sem))
            fills.append(pltpu.async_copy(
                row_hbm.at[pl.ds(wid, 1)],
                plane_v.at[pl.ds(r, 1), pl.ds(d, d)], sem))
        for cpy in fills:
            cpy.wait()
        pltpu.sync_copy(plane_v, plane_hbm.at[pl.ds(wid * rows, rows), :])

    return pos_plane_kernel(row_embed, col_embed)


def _tc_broadcast(plane, nt):
    hw, c2 = plane.shape

    k = 4

    def body(plane_hbm, out_hbm, vbuf, sem_in, sem_out):
        pltpu.async_copy(plane_hbm, vbuf.at[0], sem_in).wait()
        pltpu.async_copy(vbuf.at[pl.ds(0, 1)], vbuf.at[pl.ds(1, 1)],
                         sem_in).wait()
        pltpu.async_copy(vbuf.at[pl.ds(0, 2)], vbuf.at[pl.ds(2, 2)],
                         sem_in).wait()
        outs = [pltpu.async_copy(vbuf, out_hbm.at[pl.ds(g * k, k)], sem_out)
                for g in range(nt // k)]
        for cpy in outs:
            cpy.wait()

    return pl.pallas_call(
        body,
        in_specs=[pl.BlockSpec(memory_space=pltpu.MemorySpace.HBM)],
        out_specs=pl.BlockSpec(memory_space=pltpu.MemorySpace.HBM),
        out_shape=jax.ShapeDtypeStruct((nt, hw, c2), jnp.float32),
        scratch_shapes=[
            pltpu.VMEM((k, hw, c2), jnp.float32),
            pltpu.SemaphoreType.DMA,
            pltpu.SemaphoreType.DMA,
        ],
    )(plane)


def kernel(tensor_list, row_embed, col_embed):
    nt, f, _ = tensor_list.shape
    side = int(f ** 0.5)
    d = row_embed.shape[1]
    assert col_embed.shape[1] == d
    plane = _sc_plane(row_embed, col_embed, side * side, d)
    return _tc_broadcast(plane, nt)


# speedup vs baseline: 1.0196x; 1.0196x over previous
import functools

import jax
import jax.numpy as jnp
from jax import lax
from jax.experimental import pallas as pl
from jax.experimental.pallas import tpu as pltpu
from jax.experimental.pallas import tpu_sc as plsc


def _sc_plane(row_embed, col_embed, hw, d):
    info = plsc.get_sparse_core_info()
    nc, ns = info.num_cores, info.num_subcores
    nw = nc * ns
    rows = hw // nw
    mesh = plsc.VectorSubcoreMesh(core_axis_name="c", subcore_axis_name="s")

    @functools.partial(
        pl.kernel,
        out_type=jax.ShapeDtypeStruct((hw, 2 * d), jnp.float32),
        mesh=mesh,
        scratch_types=[
            pltpu.VMEM((rows, 2 * d), jnp.float32),
            pltpu.SemaphoreType.DMA,
        ],
    )
    def pos_plane_kernel(row_hbm, col_hbm, plane_hbm, plane_v, sem):
        wid = lax.axis_index("s") * nc + lax.axis_index("c")
        fills = []
        for r in range(rows):
            fills.append(pltpu.async_copy(
                col_hbm.at[pl.ds(r, 1)],
                plane_v.at[pl.ds(r, 1), pl.ds(0, d)], sem))
            fills.append(pltpu.async_copy(
                row_hbm.at[pl.ds(wid, 1)],
                plane_v.at[pl.ds(r, 1), pl.ds(d, d)], sem))
        for cpy in fills:
            cpy.wait()
        pltpu.sync_copy(plane_v, plane_hbm.at[pl.ds(wid * rows, rows), :])

    return pos_plane_kernel(row_embed, col_embed)


def _tc_broadcast(plane, nt):
    hw, c2 = plane.shape

    k = 4

    def body(plane_ref, out_ref):
        out_ref[...] = jnp.broadcast_to(plane_ref[...][None], (k, hw, c2))

    return pl.pallas_call(
        body,
        grid=(nt // k,),
        in_specs=[pl.BlockSpec((hw, c2), lambda i: (0, 0))],
        out_specs=pl.BlockSpec((k, hw, c2), lambda i: (i, 0, 0)),
        out_shape=jax.ShapeDtypeStruct((nt, hw, c2), jnp.float32),
    )(plane)


def kernel(tensor_list, row_embed, col_embed):
    nt, f, _ = tensor_list.shape
    side = int(f ** 0.5)
    d = row_embed.shape[1]
    assert col_embed.shape[1] == d
    plane = _sc_plane(row_embed, col_embed, side * side, d)
    return _tc_broadcast(plane, nt)


# speedup vs baseline: 1.0600x; 1.0396x over previous
import functools

import jax
import jax.numpy as jnp
from jax import lax
from jax.experimental import pallas as pl
from jax.experimental.pallas import tpu as pltpu
from jax.experimental.pallas import tpu_sc as plsc


def _sc_plane(row_embed, col_embed, hw, d):
    info = plsc.get_sparse_core_info()
    nc, ns = info.num_cores, info.num_subcores
    nw = nc * ns
    rows = hw // nw
    mesh = plsc.VectorSubcoreMesh(core_axis_name="c", subcore_axis_name="s")

    @functools.partial(
        pl.kernel,
        out_type=jax.ShapeDtypeStruct((hw, 2 * d), jnp.float32),
        mesh=mesh,
        scratch_types=[
            pltpu.VMEM((rows, 2 * d), jnp.float32),
            pltpu.SemaphoreType.DMA,
        ],
    )
    def pos_plane_kernel(row_hbm, col_hbm, plane_hbm, plane_v, sem):
        wid = lax.axis_index("s") * nc + lax.axis_index("c")
        col_cp = pltpu.async_copy(
            col_hbm.at[pl.ds(0, rows)],
            plane_v.at[:, pl.ds(0, d)], sem)
        pltpu.async_copy(
            row_hbm.at[pl.ds(wid, 1)],
            plane_v.at[pl.ds(0, 1), pl.ds(d, d)], sem).wait()
        lanes = 16
        for c in range(d // lanes):
            v = plane_v[0, pl.ds(d + c * lanes, lanes)]
            for r in range(1, rows):
                plane_v[r, pl.ds(d + c * lanes, lanes)] = v
        col_cp.wait()
        pltpu.sync_copy(plane_v, plane_hbm.at[pl.ds(wid * rows, rows), :])

    return pos_plane_kernel(row_embed, col_embed)


def _tc_broadcast(plane, nt):
    hw, c2 = plane.shape

    k = 8

    def body(plane_ref, out_ref):
        out_ref[...] = jnp.broadcast_to(plane_ref[...][None], (k, hw, c2))

    return pl.pallas_call(
        body,
        grid=(nt // k,),
        in_specs=[pl.BlockSpec((hw, c2), lambda i: (0, 0))],
        out_specs=pl.BlockSpec((k, hw, c2), lambda i: (i, 0, 0)),
        out_shape=jax.ShapeDtypeStruct((nt, hw, c2), jnp.float32),
    )(plane)


def kernel(tensor_list, row_embed, col_embed):
    nt, f, _ = tensor_list.shape
    side = int(f ** 0.5)
    d = row_embed.shape[1]
    assert col_embed.shape[1] == d
    plane = _sc_plane(row_embed, col_embed, side * side, d)
    return _tc_broadcast(plane, nt)
